# baseline (speedup 1.0000x reference)
import jax
import jax.numpy as jnp
from jax import lax
from jax.experimental import pallas as pl
from jax.experimental.pallas import tpu as pltpu
from jax.experimental.pallas import tpu_sc as plsc

N_USERS = 10000
N_ITEMS = 40000
N = N_USERS + N_ITEMS
E = 800000
D = 64

NC = 2
NS = 16
H = N // NC
TRASH = H
ACC_ROWS = H + 88
B = 128
NBT = 6272
E_PAD = NBT * B
NB = NBT // NS
REC = 3 * B

Z_PER_TILE = ACC_ROWS // NS
CP_PER_TILE = 1560


def _bcast_lane(v16, e):
    idx = jnp.full((16, 1), e, jnp.int32)
    return lax.gather(
        v16, idx,
        dimension_numbers=lax.GatherDimensionNumbers(
            offset_dims=(), collapsed_slice_dims=(0,), start_index_map=(0,)),
        slice_sizes=(1,),
        mode=lax.GatherScatterMode.PROMISE_IN_BOUNDS)


def _layer_body(x, pk, y,
                pkb0, pkb1, rowsb0, rowsb1, lidxb0, lidxb1, zbuf, acc,
                stg0, stg1, gat0, gat1, sct0, sct1):
    c = lax.axis_index("c")
    s = lax.axis_index("s")
    base_dst = c * H

    pkb = (pkb0, pkb1)
    rowsb = (rowsb0, rowsb1)
    lidxb = (lidxb0, lidxb1)
    stg = (stg0, stg1)
    gat = (gat0, gat1)
    sct = (sct0, sct1)

    def zrow(r, _):
        for k in range(4):
            zbuf[r, pl.ds(k * 16, 16)] = jnp.zeros((16,), jnp.float32)
        return 0
    lax.fori_loop(0, 32, zrow, 0)
    z0 = s * Z_PER_TILE
    def zcopy(i, _):
        pltpu.sync_copy(zbuf, acc.at[pl.ds(z0 + i * 32, 32)])
        return 0
    lax.fori_loop(0, Z_PER_TILE // 32, zcopy, 0)
    plsc.subcore_barrier()

    def stage(bidx, p):
        pltpu.async_copy(pk.at[pl.ds(bidx * REC, REC)], pkb[p], stg[p])

    def wait_stage(p):
        pltpu.make_async_copy(pk.at[pl.ds(0, REC)], pkb[p], stg[p]).wait()

    def gather(p):
        pltpu.async_copy(x.at[pkb[p].at[pl.ds(0, B)]], rowsb[p], gat[p])

    def wait_gather(p):
        pltpu.make_async_copy(x.at[pkb[p].at[pl.ds(0, B)]], rowsb[p],
                              gat[p]).wait()

    def scatter(p):
        pltpu.async_copy(rowsb[p], acc.at[lidxb[p]], sct[p], add=True)

    def wait_scatter(p):
        pltpu.make_async_copy(rowsb[p], acc.at[lidxb[p]], sct[p]).wait()

    def compute(p):
        for g in range(B // 16):
            gb = g * 16
            d16 = pkb[p][pl.ds(B + gb, 16)]
            inr = (d16 >= base_dst) & (d16 < base_dst + H)
            lidxb[p][pl.ds(gb, 16)] = jnp.where(inr, d16 - base_dst, TRASH)
            v16 = lax.bitcast_convert_type(
                pkb[p][pl.ds(2 * B + gb, 16)], jnp.float32)
            for e in range(16):
                sv = _bcast_lane(v16, e)
                r = gb + e
                for k in range(4):
                    rowsb[p][r, pl.ds(k * 16, 16)] = (
                        rowsb[p][r, pl.ds(k * 16, 16)] * sv)

    stage(s, 0)
    stage(s + NS, 1)
    wait_stage(0)
    gather(0)

    def pair(i, _):
        wait_gather(0)
        compute(0)
        scatter(0)
        stage(s + (2 * i + 2) * NS, 0)
        @pl.when(i > 0)
        def _():
            wait_scatter(1)
        wait_stage(1)
        gather(1)
        wait_gather(1)
        compute(1)
        scatter(1)
        stage(s + (2 * i + 3) * NS, 1)
        wait_scatter(0)
        wait_stage(0)
        @pl.when(i < NB // 2 - 1)
        def _():
            gather(0)
        return 0
    lax.fori_loop(0, NB // 2, pair, 0)

    wait_scatter(1)
    wait_stage(1)

    plsc.subcore_barrier()
    r0 = s * CP_PER_TILE
    pltpu.sync_copy(acc.at[pl.ds(r0, CP_PER_TILE)],
                    y.at[pl.ds(base_dst + r0, CP_PER_TILE)])
    @pl.when(s == 0)
    def _():
        pltpu.sync_copy(acc.at[pl.ds(NS * CP_PER_TILE, 40)],
                        y.at[pl.ds(base_dst + NS * CP_PER_TILE, 40)])


def _sc_layer(x, pk):
    mesh = plsc.VectorSubcoreMesh(
        core_axis_name="c", subcore_axis_name="s",
        num_cores=NC, num_subcores=NS)
    return pl.kernel(
        _layer_body,
        out_type=jax.ShapeDtypeStruct((N, D), jnp.float32),
        mesh=mesh,
        compiler_params=pltpu.CompilerParams(use_tc_tiling_on_sc=False),
        scratch_types=[
            pltpu.VMEM((REC,), jnp.int32),
            pltpu.VMEM((REC,), jnp.int32),
            pltpu.VMEM((B, D), jnp.float32),
            pltpu.VMEM((B, D), jnp.float32),
            pltpu.VMEM((B,), jnp.int32),
            pltpu.VMEM((B,), jnp.int32),
            pltpu.VMEM((32, D), jnp.float32),
            pltpu.VMEM_SHARED((ACC_ROWS, D), jnp.float32),
            pltpu.SemaphoreType.DMA,
            pltpu.SemaphoreType.DMA,
            pltpu.SemaphoreType.DMA,
            pltpu.SemaphoreType.DMA,
            pltpu.SemaphoreType.DMA,
            pltpu.SemaphoreType.DMA,
        ],
    )(x, pk)


def _mean_body(a, b, c, d, o):
    o[...] = (a[...] + b[...] + c[...] + d[...]) * 0.25


def _mean4(x0, x1, x2, x3):
    xs = [v.reshape(N // 2, 2 * D) for v in (x0, x1, x2, x3)]
    spec = pl.BlockSpec((5000, 2 * D), lambda i: (i, 0))
    out = pl.pallas_call(
        _mean_body,
        grid=(5,),
        in_specs=[spec] * 4,
        out_specs=spec,
        out_shape=jax.ShapeDtypeStruct((N // 2, 2 * D), jnp.float32),
    )(*xs)
    return out.reshape(N, D)


def kernel(user_emb, item_emb, adj_vals, adj_rows, adj_cols):
    x0 = jnp.concatenate([user_emb, item_emb], axis=0)
    pad = E_PAD - E
    zi = jnp.zeros((pad,), jnp.int32)
    ca = jnp.concatenate([adj_cols, zi]).reshape(NBT, B)
    ra = jnp.concatenate([adj_rows, zi]).reshape(NBT, B)
    va = jnp.concatenate(
        [lax.bitcast_convert_type(adj_vals, jnp.int32), zi]).reshape(NBT, B)
    pk = jnp.stack([ca, ra, va], axis=1).reshape(NBT * REC)
    x1 = _sc_layer(x0, pk)
    x2 = _sc_layer(x1, pk)
    x3 = _sc_layer(x2, pk)
    out = _mean4(x0, x1, x2, x3)
    return (out[:N_USERS], out[N_USERS:])
